# baseline (speedup 1.0000x reference)
import functools

import jax
import jax.numpy as jnp
from jax import lax
from jax.experimental import pallas as pl
from jax.experimental.pallas import tpu as pltpu, tpu_sc as plsc

K = 1024
D = 512
BN = 256
SEG = 3


def _argmin_kernel(z_ref, cbt_ref, idx_ref):
    zb = z_ref[...]
    cbt = cbt_ref[...]
    dots = jax.lax.dot_general(
        zb, cbt, (((1,), (0,)), ((), ())),
        preferred_element_type=jnp.float32)
    e_sq = jnp.sum(cbt * cbt, axis=0, keepdims=True)
    scores = e_sq - 2.0 * dots
    idx_ref[0, 0, :] = jnp.argmin(scores, axis=1).astype(jnp.int32)


def _tc_indices(zf, codebook_t):
    n = zf.shape[0]
    nb = n // BN
    idx = pl.pallas_call(
        _argmin_kernel,
        grid=(nb,),
        in_specs=[
            pl.BlockSpec((BN, D), lambda i: (i, 0)),
            pl.BlockSpec((D, K), lambda i: (0, 0)),
        ],
        out_specs=pl.BlockSpec((1, 1, BN), lambda i: (i, 0, 0)),
        out_shape=jax.ShapeDtypeStruct((nb, 1, BN), jnp.int32),
    )(zf, codebook_t)
    return idx.reshape(n)


def _make_sc_gather(n):
    info = plsc.get_sparse_core_info()
    nw = info.num_cores * info.num_subcores
    b_per_w = n // nw
    mesh = plsc.VectorSubcoreMesh(core_axis_name="c", subcore_axis_name="s")

    @functools.partial(
        pl.kernel, mesh=mesh,
        out_type=jax.ShapeDtypeStruct((n, D), jnp.float32),
        scratch_types=[
            pltpu.VMEM((b_per_w,), jnp.int32),
            pltpu.VMEM((b_per_w, D), jnp.float32),
            pltpu.SemaphoreType.DMA,
        ],
    )
    def gather(table_hbm, idx_hbm, out_hbm, idx_v, rows_v, sem):
        wid = lax.axis_index("s") * info.num_cores + lax.axis_index("c")
        base = wid * b_per_w
        pltpu.sync_copy(idx_hbm.at[pl.ds(base, b_per_w)], idx_v)
        pltpu.async_copy(table_hbm.at[idx_v], rows_v, sem).wait()
        pltpu.sync_copy(rows_v, out_hbm.at[pl.ds(base, b_per_w)])

    return gather


def kernel(z, codebook):
    B, T, Dd = z.shape
    zf = z.reshape(-1, Dd)
    n = zf.shape[0]
    seg_n = n // SEG
    sc_gather = _make_sc_gather(seg_n)
    cbt = codebook.T
    qs = []
    for s in range(SEG):
        zseg = lax.slice_in_dim(zf, s * seg_n, (s + 1) * seg_n, axis=0)
        idx = _tc_indices(zseg, cbt)
        qs.append(sc_gather(codebook, idx))
    return jnp.concatenate(qs, axis=0).reshape(B, T, Dd)


# speedup vs baseline: 2.2578x; 2.2578x over previous
import jax
import jax.numpy as jnp
from jax.experimental import pallas as pl
from jax.experimental.pallas import tpu as pltpu

K = 1024
D = 512
BN = 256


def _vq_kernel(z_ref, cbt_ref, cb_ref, out_ref, esq_ref):
    cbt = cbt_ref[...]

    @pl.when(pl.program_id(0) == 0)
    def _():
        esq_ref[...] = jnp.sum(cbt * cbt, axis=0, keepdims=True)

    zb = z_ref[...]
    dots = jax.lax.dot_general(
        zb, cbt, (((1,), (0,)), ((), ())),
        preferred_element_type=jnp.float32)
    scores = esq_ref[...] - 2.0 * dots
    idx = jnp.argmin(scores, axis=1)
    oh = (jax.lax.broadcasted_iota(jnp.int32, scores.shape, 1)
          == idx[:, None]).astype(jnp.float32)
    q = jax.lax.dot_general(
        oh, cb_ref[...], (((1,), (0,)), ((), ())),
        preferred_element_type=jnp.float32)
    out_ref[...] = q


def kernel(z, codebook):
    B, T, Dd = z.shape
    zf = z.reshape(-1, Dd)
    n = zf.shape[0]
    grid = (n // BN,)
    out = pl.pallas_call(
        _vq_kernel,
        grid=grid,
        in_specs=[
            pl.BlockSpec((BN, Dd), lambda i: (i, 0)),
            pl.BlockSpec((Dd, K), lambda i: (0, 0)),
            pl.BlockSpec((K, Dd), lambda i: (0, 0)),
        ],
        out_specs=pl.BlockSpec((BN, Dd), lambda i: (i, 0)),
        out_shape=jax.ShapeDtypeStruct((n, Dd), jnp.float32),
        scratch_shapes=[pltpu.VMEM((1, K), jnp.float32)],
    )(zf, codebook.T, codebook)
    return out.reshape(B, T, Dd)


# speedup vs baseline: 2.6942x; 1.1933x over previous
import jax
import jax.numpy as jnp
from jax.experimental import pallas as pl

K = 1024
D = 512
BN = 512


def _vq_kernel(z_ref, cbt_ref, cb_ref, out_ref):
    zb = z_ref[...]
    cbt = cbt_ref[...]
    dots = jax.lax.dot_general(
        zb, cbt, (((1,), (0,)), ((), ())),
        preferred_element_type=jnp.float32)
    e_sq = jnp.sum(cbt * cbt, axis=0, keepdims=True)
    scores = e_sq - 2.0 * dots
    idx = jnp.argmin(scores, axis=1)
    oh = (jax.lax.broadcasted_iota(jnp.int32, scores.shape, 1)
          == idx[:, None]).astype(jnp.float32)
    q = jax.lax.dot_general(
        oh, cb_ref[...], (((1,), (0,)), ((), ())),
        preferred_element_type=jnp.float32)
    out_ref[...] = zb + (q - zb)


def kernel(z, codebook):
    B, T, Dd = z.shape
    zf = z.reshape(-1, Dd)
    n = zf.shape[0]
    grid = (n // BN,)
    out = pl.pallas_call(
        _vq_kernel,
        grid=grid,
        in_specs=[
            pl.BlockSpec((BN, Dd), lambda i: (i, 0)),
            pl.BlockSpec((Dd, K), lambda i: (0, 0)),
            pl.BlockSpec((K, Dd), lambda i: (0, 0)),
        ],
        out_specs=pl.BlockSpec((BN, Dd), lambda i: (i, 0)),
        out_shape=jax.ShapeDtypeStruct((n, Dd), jnp.float32),
    )(zf, codebook.T, codebook)
    return out.reshape(B, T, Dd)


# speedup vs baseline: 3.2968x; 1.2237x over previous
import jax
import jax.numpy as jnp
from jax.experimental import pallas as pl

K = 1024
D = 512
BN = 768


def _vq_kernel(z_ref, cbt_ref, cb_ref, out_ref):
    zb = z_ref[...]
    cbt = cbt_ref[...]
    dots = jax.lax.dot_general(
        zb, cbt, (((1,), (0,)), ((), ())),
        preferred_element_type=jnp.float32)
    e_sq = jnp.sum(cbt * cbt, axis=0, keepdims=True)
    scores = e_sq - 2.0 * dots
    idx = jnp.argmin(scores, axis=1)
    oh = (jax.lax.broadcasted_iota(jnp.int32, scores.shape, 1)
          == idx[:, None]).astype(jnp.float32)
    q = jax.lax.dot_general(
        oh, cb_ref[...], (((1,), (0,)), ((), ())),
        preferred_element_type=jnp.float32)
    out_ref[...] = zb + (q - zb)


def kernel(z, codebook):
    B, T, Dd = z.shape
    zf = z.reshape(-1, Dd)
    n = zf.shape[0]
    grid = (n // BN,)
    out = pl.pallas_call(
        _vq_kernel,
        grid=grid,
        in_specs=[
            pl.BlockSpec((BN, Dd), lambda i: (i, 0)),
            pl.BlockSpec((Dd, K), lambda i: (0, 0)),
            pl.BlockSpec((K, Dd), lambda i: (0, 0)),
        ],
        out_specs=pl.BlockSpec((BN, Dd), lambda i: (i, 0)),
        out_shape=jax.ShapeDtypeStruct((n, Dd), jnp.float32),
    )(zf, codebook.T, codebook)
    return out.reshape(B, T, Dd)
